# baseline (speedup 1.0000x reference)
import jax
import jax.numpy as jnp
from jax.experimental import pallas as pl
from jax.experimental.pallas import tpu as pltpu

_TOPK = 100
_BLK = 1024
_SLOTS = 8
_TBLK = 4096
_BIG_I32 = 2**31 - 1


def _thresh_body(q_ref, k_ref, t_ref, acc_ref, nk, blk, topk):
    b = pl.program_id(0)
    nb = pl.num_programs(0)
    s = jax.lax.dot_general(
        k_ref[...], q_ref[...],
        (((1,), (1,)), ((), ())),
        preferred_element_type=jnp.float32,
    )
    nq = s.shape[1]
    liota = jax.lax.broadcasted_iota(jnp.int32, (blk, nq), 0)
    s = jnp.where(jnp.logical_or(b < nb - 1, liota + b * blk < nk),
                  s, -jnp.inf)

    @pl.when(b == 0)
    def _():
        acc_ref[...] = jnp.full(acc_ref.shape, -jnp.inf, jnp.float32)

    acc_ref[pl.ds(b, 1), :] = jnp.max(s, axis=0, keepdims=True)

    @pl.when(b == nb - 1)
    def _():
        def step(i, m):
            cur = acc_ref[...]
            m = jnp.max(cur, axis=0, keepdims=True)
            acc_ref[...] = jnp.where(cur == m, -jnp.inf, cur)
            return m
        t_ref[...] = jax.lax.fori_loop(
            0, topk, step, jnp.zeros((1, nq), jnp.float32))


def _extract_body(q_ref, k_ref, t_ref, val_ref, idx_ref, s_ref,
                  nk, blk, slots):
    b = pl.program_id(0)
    s = jax.lax.dot_general(
        k_ref[...], q_ref[...],
        (((1,), (1,)), ((), ())),
        preferred_element_type=jnp.float32,
    )
    nq = s.shape[1]
    liota = jax.lax.broadcasted_iota(jnp.int32, (blk, nq), 0)
    nb = pl.num_programs(0)
    s = jnp.where(jnp.logical_or(b < nb - 1, liota + b * blk < nk),
                  s, -jnp.inf)
    val_ref[...] = jnp.full((slots, nq), -jnp.inf, jnp.float32)
    idx_ref[...] = jnp.full((slots, nq), _BIG_I32, jnp.int32)
    t = t_ref[...]
    cnt = jnp.sum(jnp.where(s >= t, 1.0, 0.0), axis=0)
    maxcnt = jnp.max(cnt)

    def extract(cur, i):
        m = jnp.max(cur, axis=0, keepdims=True)
        lw = jnp.argmax(cur, axis=0).reshape(1, nq)
        val_ref[pl.ds(i, 1), :] = m
        idx_ref[pl.ds(i, 1), :] = lw + b * blk
        return jnp.where(liota == lw, -jnp.inf, cur)

    s_ref[...] = extract(s, 0)
    for i in range(1, slots):
        @pl.when(maxcnt > i)
        def _(i=i):
            s_ref[...] = extract(s_ref[...], i)


def _merge_body(val_ref, idx_ref, oval_ref, oidx_ref, topk):
    def step(i, carry):
        cur = val_ref[...]
        idx = idx_ref[...]
        m = jnp.max(cur, axis=0, keepdims=True)
        cand = jnp.where(cur == m, idx, _BIG_I32)
        win = jnp.min(cand, axis=0, keepdims=True)
        oval_ref[pl.ds(i, 1), :] = m
        oidx_ref[pl.ds(i, 1), :] = win
        val_ref[...] = jnp.where(idx == win, -jnp.inf, cur)
        return carry
    jax.lax.fori_loop(0, topk, step, 0)


def kernel(queries, keys):
    nq, d = queries.shape
    nk = keys.shape[0]

    nb = (nk + _BLK - 1) // _BLK
    thresh = pl.pallas_call(
        lambda q, k, t, a: _thresh_body(q, k, t, a, nk, _BLK, _TOPK),
        grid=(nb,),
        in_specs=[
            pl.BlockSpec((nq, d), lambda i: (0, 0)),
            pl.BlockSpec((_BLK, d), lambda i: (i, 0)),
        ],
        out_specs=pl.BlockSpec((1, nq), lambda i: (0, 0)),
        out_shape=jax.ShapeDtypeStruct((1, nq), jnp.float32),
        scratch_shapes=[pltpu.VMEM((8 * ((nb + 7) // 8), nq), jnp.float32)],
    )(queries, keys)
    rows = nb * _SLOTS
    cand_val, cand_idx = pl.pallas_call(
        lambda q, k, t, v, x, s: _extract_body(q, k, t, v, x, s,
                                               nk, _BLK, _SLOTS),
        grid=(nb,),
        in_specs=[
            pl.BlockSpec((nq, d), lambda i: (0, 0)),
            pl.BlockSpec((_BLK, d), lambda i: (i, 0)),
            pl.BlockSpec((1, nq), lambda i: (0, 0)),
        ],
        out_specs=[
            pl.BlockSpec((_SLOTS, nq), lambda i: (i, 0)),
            pl.BlockSpec((_SLOTS, nq), lambda i: (i, 0)),
        ],
        out_shape=[
            jax.ShapeDtypeStruct((rows, nq), jnp.float32),
            jax.ShapeDtypeStruct((rows, nq), jnp.int32),
        ],
        scratch_shapes=[pltpu.VMEM((_BLK, nq), jnp.float32)],
    )(queries, keys, thresh)

    pad = 8 * ((_TOPK + 7) // 8)
    top_val, top_idx = pl.pallas_call(
        lambda v, x, ov, ox: _merge_body(v, x, ov, ox, _TOPK),
        in_specs=[
            pl.BlockSpec((rows, nq), lambda: (0, 0)),
            pl.BlockSpec((rows, nq), lambda: (0, 0)),
        ],
        out_specs=[
            pl.BlockSpec((pad, nq), lambda: (0, 0)),
            pl.BlockSpec((pad, nq), lambda: (0, 0)),
        ],
        out_shape=[
            jax.ShapeDtypeStruct((pad, nq), jnp.float32),
            jax.ShapeDtypeStruct((pad, nq), jnp.int32),
        ],
    )(cand_val, cand_idx)

    return top_val[:_TOPK].T, top_idx[:_TOPK].T


# speedup vs baseline: 1.2260x; 1.2260x over previous
import jax
import jax.numpy as jnp
from jax.experimental import pallas as pl
from jax.experimental.pallas import tpu as pltpu

_TOPK = 100
_BLK = 1024
_SLOTS = 8
_TBLK = 4096
_BIG_I32 = 2**31 - 1


def _thresh_body(q_ref, k_ref, t_ref, acc_ref, nk, blk, topk):
    b = pl.program_id(0)
    nb = pl.num_programs(0)
    s = jax.lax.dot_general(
        k_ref[...], q_ref[...],
        (((1,), (1,)), ((), ())),
        preferred_element_type=jnp.float32,
    )
    nq = s.shape[1]
    liota = jax.lax.broadcasted_iota(jnp.int32, (blk, nq), 0)
    s = jnp.where(jnp.logical_or(b < nb - 1, liota + b * blk < nk),
                  s, -jnp.inf)

    @pl.when(b == 0)
    def _():
        acc_ref[...] = jnp.full(acc_ref.shape, -jnp.inf, jnp.float32)

    sub = min(1024, blk)
    nsub = blk // sub
    acc_ref[pl.ds(b * nsub, nsub), :] = jnp.max(
        s.reshape(nsub, sub, nq), axis=1)

    @pl.when(b == nb - 1)
    def _():
        def step(i, m):
            cur = acc_ref[...]
            m = jnp.max(cur, axis=0, keepdims=True)
            acc_ref[...] = jnp.where(cur == m, -jnp.inf, cur)
            return m
        t_ref[...] = jax.lax.fori_loop(
            0, topk, step, jnp.zeros((1, nq), jnp.float32))


def _extract_body(q_ref, k_ref, t_ref, val_ref, idx_ref, s_ref,
                  nk, blk, slots):
    b = pl.program_id(0)
    s = jax.lax.dot_general(
        k_ref[...], q_ref[...],
        (((1,), (1,)), ((), ())),
        preferred_element_type=jnp.float32,
    )
    nq = s.shape[1]
    liota = jax.lax.broadcasted_iota(jnp.int32, (blk, nq), 0)
    nb = pl.num_programs(0)
    s = jnp.where(jnp.logical_or(b < nb - 1, liota + b * blk < nk),
                  s, -jnp.inf)
    val_ref[...] = jnp.full((slots, nq), -jnp.inf, jnp.float32)
    idx_ref[...] = jnp.full((slots, nq), _BIG_I32, jnp.int32)
    t = t_ref[...]
    cnt = jnp.sum(jnp.where(s >= t, 1.0, 0.0), axis=0)
    maxcnt = jnp.max(cnt)

    def extract(cur, i):
        m = jnp.max(cur, axis=0, keepdims=True)
        lw = jnp.argmax(cur, axis=0).reshape(1, nq)
        val_ref[pl.ds(i, 1), :] = m
        idx_ref[pl.ds(i, 1), :] = lw + b * blk
        return jnp.where(liota == lw, -jnp.inf, cur)

    s_ref[...] = extract(s, 0)
    for i in range(1, slots):
        @pl.when(maxcnt > i)
        def _(i=i):
            s_ref[...] = extract(s_ref[...], i)


def _merge_body(val_ref, idx_ref, oval_ref, oidx_ref, topk):
    def step(i, carry):
        cur = val_ref[...]
        idx = idx_ref[...]
        m = jnp.max(cur, axis=0, keepdims=True)
        cand = jnp.where(cur == m, idx, _BIG_I32)
        win = jnp.min(cand, axis=0, keepdims=True)
        oval_ref[pl.ds(i, 1), :] = m
        oidx_ref[pl.ds(i, 1), :] = win
        val_ref[...] = jnp.where(idx == win, -jnp.inf, cur)
        return carry
    jax.lax.fori_loop(0, topk, step, 0)


def kernel(queries, keys):
    nq, d = queries.shape
    nk = keys.shape[0]

    nb = (nk + _BLK - 1) // _BLK
    nb0 = (nk + _TBLK - 1) // _TBLK
    nacc = nb0 * (_TBLK // min(1024, _TBLK))
    thresh = pl.pallas_call(
        lambda q, k, t, a: _thresh_body(q, k, t, a, nk, _TBLK, _TOPK),
        grid=(nb0,),
        in_specs=[
            pl.BlockSpec((nq, d), lambda i: (0, 0)),
            pl.BlockSpec((_TBLK, d), lambda i: (i, 0)),
        ],
        out_specs=pl.BlockSpec((1, nq), lambda i: (0, 0)),
        out_shape=jax.ShapeDtypeStruct((1, nq), jnp.float32),
        scratch_shapes=[pltpu.VMEM((8 * ((nacc + 7) // 8), nq), jnp.float32)],
    )(queries, keys)
    rows = nb * _SLOTS
    cand_val, cand_idx = pl.pallas_call(
        lambda q, k, t, v, x, s: _extract_body(q, k, t, v, x, s,
                                               nk, _BLK, _SLOTS),
        grid=(nb,),
        in_specs=[
            pl.BlockSpec((nq, d), lambda i: (0, 0)),
            pl.BlockSpec((_BLK, d), lambda i: (i, 0)),
            pl.BlockSpec((1, nq), lambda i: (0, 0)),
        ],
        out_specs=[
            pl.BlockSpec((_SLOTS, nq), lambda i: (i, 0)),
            pl.BlockSpec((_SLOTS, nq), lambda i: (i, 0)),
        ],
        out_shape=[
            jax.ShapeDtypeStruct((rows, nq), jnp.float32),
            jax.ShapeDtypeStruct((rows, nq), jnp.int32),
        ],
        scratch_shapes=[pltpu.VMEM((_BLK, nq), jnp.float32)],
    )(queries, keys, thresh)

    pad = 8 * ((_TOPK + 7) // 8)
    top_val, top_idx = pl.pallas_call(
        lambda v, x, ov, ox: _merge_body(v, x, ov, ox, _TOPK),
        in_specs=[
            pl.BlockSpec((rows, nq), lambda: (0, 0)),
            pl.BlockSpec((rows, nq), lambda: (0, 0)),
        ],
        out_specs=[
            pl.BlockSpec((pad, nq), lambda: (0, 0)),
            pl.BlockSpec((pad, nq), lambda: (0, 0)),
        ],
        out_shape=[
            jax.ShapeDtypeStruct((pad, nq), jnp.float32),
            jax.ShapeDtypeStruct((pad, nq), jnp.int32),
        ],
    )(cand_val, cand_idx)

    return top_val[:_TOPK].T, top_idx[:_TOPK].T


# speedup vs baseline: 1.5334x; 1.2507x over previous
import jax
import jax.numpy as jnp
from jax.experimental import pallas as pl
from jax.experimental.pallas import tpu as pltpu

_TOPK = 100
_BLK = 2048
_SLOTS = 8
_TBLK = 4096
_BIG_I32 = 2**31 - 1


def _thresh_body(q_ref, k_ref, t_ref, acc_ref, nk, blk, topk):
    b = pl.program_id(0)
    nb = pl.num_programs(0)
    s = jax.lax.dot_general(
        k_ref[...], q_ref[...],
        (((1,), (1,)), ((), ())),
        preferred_element_type=jnp.float32,
    )
    nq = s.shape[1]
    liota = jax.lax.broadcasted_iota(jnp.int32, (blk, nq), 0)
    s = jnp.where(jnp.logical_or(b < nb - 1, liota + b * blk < nk),
                  s, -jnp.inf)

    @pl.when(b == 0)
    def _():
        acc_ref[...] = jnp.full(acc_ref.shape, -jnp.inf, jnp.float32)

    sub = min(1024, blk)
    nsub = blk // sub
    acc_ref[pl.ds(b * nsub, nsub), :] = jnp.max(
        s.reshape(nsub, sub, nq), axis=1)

    @pl.when(b == nb - 1)
    def _():
        def step(i, m):
            cur = acc_ref[...]
            m = jnp.max(cur, axis=0, keepdims=True)
            acc_ref[...] = jnp.where(cur == m, -jnp.inf, cur)
            return m
        t_ref[...] = jax.lax.fori_loop(
            0, topk, step, jnp.zeros((1, nq), jnp.float32))


def _extract_body(q_ref, k_ref, t_ref, val_ref, idx_ref, s_ref,
                  nk, blk, slots):
    b = pl.program_id(0)
    s = jax.lax.dot_general(
        k_ref[...], q_ref[...],
        (((1,), (1,)), ((), ())),
        preferred_element_type=jnp.float32,
    )
    nq = s.shape[1]
    liota = jax.lax.broadcasted_iota(jnp.int32, (blk, nq), 0)
    nb = pl.num_programs(0)
    s = jnp.where(jnp.logical_or(b < nb - 1, liota + b * blk < nk),
                  s, -jnp.inf)
    val_ref[...] = jnp.full((slots, nq), -jnp.inf, jnp.float32)
    idx_ref[...] = jnp.full((slots, nq), _BIG_I32, jnp.int32)
    t = t_ref[...]
    cnt = jnp.sum(jnp.where(s >= t, 1.0, 0.0), axis=0)
    maxcnt = jnp.max(cnt)

    def extract(cur, i):
        m = jnp.max(cur, axis=0, keepdims=True)
        lw = jnp.argmax(cur, axis=0).reshape(1, nq)
        val_ref[pl.ds(i, 1), :] = m
        idx_ref[pl.ds(i, 1), :] = lw + b * blk
        return jnp.where(liota == lw, -jnp.inf, cur)

    s_ref[...] = extract(s, 0)
    for i in range(1, slots):
        @pl.when(maxcnt > i)
        def _(i=i):
            s_ref[...] = extract(s_ref[...], i)


def _merge_body(val_ref, idx_ref, oval_ref, oidx_ref, topk):
    def step(i, carry):
        cur = val_ref[...]
        idx = idx_ref[...]
        m = jnp.max(cur, axis=0, keepdims=True)
        cand = jnp.where(cur == m, idx, _BIG_I32)
        win = jnp.min(cand, axis=0, keepdims=True)
        oval_ref[pl.ds(i, 1), :] = m
        oidx_ref[pl.ds(i, 1), :] = win
        val_ref[...] = jnp.where(idx == win, -jnp.inf, cur)
        return carry
    jax.lax.fori_loop(0, topk, step, 0)


def kernel(queries, keys):
    nq, d = queries.shape
    nk = keys.shape[0]

    nb = (nk + _BLK - 1) // _BLK
    nb0 = (nk + _TBLK - 1) // _TBLK
    nacc = nb0 * (_TBLK // min(1024, _TBLK))
    thresh = pl.pallas_call(
        lambda q, k, t, a: _thresh_body(q, k, t, a, nk, _TBLK, _TOPK),
        grid=(nb0,),
        in_specs=[
            pl.BlockSpec((nq, d), lambda i: (0, 0)),
            pl.BlockSpec((_TBLK, d), lambda i: (i, 0)),
        ],
        out_specs=pl.BlockSpec((1, nq), lambda i: (0, 0)),
        out_shape=jax.ShapeDtypeStruct((1, nq), jnp.float32),
        scratch_shapes=[pltpu.VMEM((8 * ((nacc + 7) // 8), nq), jnp.float32)],
    )(queries, keys)
    rows = nb * _SLOTS
    cand_val, cand_idx = pl.pallas_call(
        lambda q, k, t, v, x, s: _extract_body(q, k, t, v, x, s,
                                               nk, _BLK, _SLOTS),
        grid=(nb,),
        in_specs=[
            pl.BlockSpec((nq, d), lambda i: (0, 0)),
            pl.BlockSpec((_BLK, d), lambda i: (i, 0)),
            pl.BlockSpec((1, nq), lambda i: (0, 0)),
        ],
        out_specs=[
            pl.BlockSpec((_SLOTS, nq), lambda i: (i, 0)),
            pl.BlockSpec((_SLOTS, nq), lambda i: (i, 0)),
        ],
        out_shape=[
            jax.ShapeDtypeStruct((rows, nq), jnp.float32),
            jax.ShapeDtypeStruct((rows, nq), jnp.int32),
        ],
        scratch_shapes=[pltpu.VMEM((_BLK, nq), jnp.float32)],
    )(queries, keys, thresh)

    pad = 8 * ((_TOPK + 7) // 8)
    top_val, top_idx = pl.pallas_call(
        lambda v, x, ov, ox: _merge_body(v, x, ov, ox, _TOPK),
        in_specs=[
            pl.BlockSpec((rows, nq), lambda: (0, 0)),
            pl.BlockSpec((rows, nq), lambda: (0, 0)),
        ],
        out_specs=[
            pl.BlockSpec((pad, nq), lambda: (0, 0)),
            pl.BlockSpec((pad, nq), lambda: (0, 0)),
        ],
        out_shape=[
            jax.ShapeDtypeStruct((pad, nq), jnp.float32),
            jax.ShapeDtypeStruct((pad, nq), jnp.int32),
        ],
    )(cand_val, cand_idx)

    return top_val[:_TOPK].T, top_idx[:_TOPK].T
